# baseline (speedup 1.0000x reference)
import jax
import jax.numpy as jnp
from jax import lax
from jax.experimental import pallas as pl
from jax.experimental.pallas import tpu as pltpu
from jax.experimental.pallas import tpu_sc as plsc

B = 16384
EMB = 64
NCATE = 5
NC = 2
NS = 16
NW = NC * NS
BPW = B // NW
CH = 128
KCH = BPW // CH
RPW = BPW // CH
UCH = 64
UKCH = BPW // UCH


def _bc_body(bid2, cid2, block_hbm, cat_hbm, out_b, out_c,
             bid_v, cid_v, b_v, c_v, sem_i, sem_b, sem_c, sem_o):
    c = lax.axis_index("c")
    s = lax.axis_index("s")
    w = s * NC + c
    base = w * BPW

    idx_cp = [pltpu.async_copy(bid2.at[pl.ds(w * RPW, RPW)], bid_v, sem_i)]
    for j in range(NCATE):
        idx_cp.append(pltpu.async_copy(
            cid2.at[pl.ds(j * (B // CH) + w * RPW, RPW)], cid_v.at[j], sem_i))
    zero = jnp.zeros((16,), jnp.float32)
    for r in range(CH):
        for q in range(2 * EMB // 16):
            c_v[0, r, pl.ds(16 * q, 16)] = zero
            c_v[1, r, pl.ds(16 * q, 16)] = zero
    for d in idx_cp:
        d.wait()

    b_cp = []
    cat_cp = [[], []]
    for k in range(2):
        b_cp.append(pltpu.async_copy(
            block_hbm.at[bid_v.at[k]], b_v.at[k], sem_b))
        for j in range(NCATE):
            cat_cp[k].append(pltpu.async_copy(
                cat_hbm.at[cid_v.at[j, k]], c_v.at[k], sem_c, add=True))
    o_cp = []
    for k in range(KCH):
        rows = pl.ds(base + k * CH, CH)
        b_cp[k].wait()
        o_cp.append(pltpu.async_copy(b_v.at[k % 2], out_b.at[rows], sem_o))
        for d in cat_cp[k % 2]:
            d.wait()
        o_cp.append(pltpu.async_copy(c_v.at[k % 2], out_c.at[rows], sem_o))
        if k + 2 < KCH:
            o_cp[2 * k].wait()
            o_cp[2 * k + 1].wait()
            for r in range(CH):
                for q in range(2 * EMB // 16):
                    c_v[k % 2, r, pl.ds(16 * q, 16)] = zero
            b_cp.append(pltpu.async_copy(
                block_hbm.at[bid_v.at[k + 2]], b_v.at[k % 2], sem_b))
            cat_cp[k % 2] = []
            for j in range(NCATE):
                cat_cp[k % 2].append(pltpu.async_copy(
                    cat_hbm.at[cid_v.at[j, k + 2]], c_v.at[k % 2], sem_c,
                    add=True))
    for k in range(2 * KCH - 4, 2 * KCH):
        o_cp[k].wait()


def _u_body(users2, uid2, out_u, uid_v, g_v, ublk, sem_i, sem_u, sem_o):
    c = lax.axis_index("c")
    s = lax.axis_index("s")
    w = s * NC + c
    base = w * BPW

    pltpu.async_copy(uid2.at[pl.ds(w * RPW, RPW)], uid_v, sem_i).wait()
    for gg in range(BPW // 16):
        f = 16 * gg
        g_v[f // UCH, pl.ds(f % UCH, 16)] = uid_v[f // CH, pl.ds(f % CH, 16)]

    u_cp = []
    for k in range(2):
        u_cp.append(pltpu.async_copy(
            users2.at[g_v.at[k]], ublk.at[k % 2], sem_u))
    o_cp = []
    for k in range(UKCH):
        u_cp[k].wait()
        o_cp.append(pltpu.async_copy(
            ublk.at[k % 2], out_u.at[pl.ds(base + k * UCH, UCH)], sem_o))
        if k + 2 < UKCH:
            o_cp[k].wait()
            u_cp.append(pltpu.async_copy(
                users2.at[g_v.at[k + 2]], ublk.at[k % 2], sem_u))
    for k in range(UKCH - 2, UKCH):
        o_cp[k].wait()


_PARAMS = pltpu.CompilerParams(use_tc_tiling_on_sc=False,
                               needs_layout_passes=False)


@jax.jit
def _din_sc(users2, uid2, bid2, cid2, blockp, catp):
    mesh = plsc.VectorSubcoreMesh(core_axis_name="c", subcore_axis_name="s",
                                  num_cores=NC, num_subcores=NS)
    out_t = jax.ShapeDtypeStruct((B, 2 * EMB), jnp.float32)
    b, cc = pl.kernel(
        _bc_body,
        out_type=(out_t, out_t),
        mesh=mesh,
        compiler_params=_PARAMS,
        scratch_types=[
            pltpu.VMEM((KCH, CH), jnp.int32),
            pltpu.VMEM((NCATE, KCH, CH), jnp.int32),
            pltpu.VMEM((2, CH, 2 * EMB), jnp.float32),
            pltpu.VMEM((2, CH, 2 * EMB), jnp.float32),
            pltpu.SemaphoreType.DMA,
            pltpu.SemaphoreType.DMA,
            pltpu.SemaphoreType.DMA,
            pltpu.SemaphoreType.DMA,
        ],
    )(bid2, cid2, blockp, catp)
    u128 = pl.kernel(
        _u_body,
        out_type=jax.ShapeDtypeStruct((B, 2 * EMB), jnp.float32),
        mesh=mesh,
        compiler_params=_PARAMS,
        scratch_types=[
            pltpu.VMEM((KCH, CH), jnp.int32),
            pltpu.VMEM((UKCH, UCH), jnp.int32),
            pltpu.VMEM((2, UCH, 2 * EMB), jnp.float32),
            pltpu.SemaphoreType.DMA,
            pltpu.SemaphoreType.DMA,
            pltpu.SemaphoreType.DMA,
        ],
    )(users2, uid2)
    return u128, b, cc


def kernel(user_id, block_id, cate_idx, users_table, block_table,
           category_table):
    users2 = jnp.pad(users_table, ((0, 0), (0, EMB)))
    uid2 = user_id.astype(jnp.int32).reshape(B // CH, CH)
    bid2 = block_id.astype(jnp.int32).reshape(B // CH, CH)
    cid2 = cate_idx.astype(jnp.int32).T.reshape(NCATE * (B // CH), CH)
    blockp = jnp.pad(block_table, ((0, 28), (0, EMB)))
    catp = jnp.pad(category_table * (1.0 / NCATE), ((0, 0), (0, EMB)))
    u128, b128, c128 = _din_sc(users2, uid2, bid2, cid2, blockp, catp)
    return jnp.concatenate(
        [u128[:, :EMB], b128[:, :EMB], c128[:, :EMB]], axis=1)


# speedup vs baseline: 1.0967x; 1.0967x over previous
import jax
import jax.numpy as jnp
from jax import lax
from jax.experimental import pallas as pl
from jax.experimental.pallas import tpu as pltpu
from jax.experimental.pallas import tpu_sc as plsc

B = 16384
EMB = 64
NCATE = 5
NC = 2
NS = 16
NW = NC * NS
BPW = B // NW
CH = 128
KCH = BPW // CH
RPW = BPW // CH
UCH = 64
UKCH = BPW // UCH


def _bc_body(bid2, cid2, block_hbm, cat_hbm, out_b, out_c,
             bid_v, cid_v, b_v, c_v, sem_i, sem_b, sem_c):
    c = lax.axis_index("c")
    s = lax.axis_index("s")
    w = s * NC + c
    base = w * BPW

    idx_cp = [pltpu.async_copy(bid2.at[pl.ds(w * RPW, RPW)], bid_v, sem_i)]
    for j in range(NCATE):
        idx_cp.append(pltpu.async_copy(
            cid2.at[pl.ds(j * (B // CH) + w * RPW, RPW)], cid_v.at[j], sem_i))
    zero = jnp.zeros((16,), jnp.float32)
    for r in range(BPW):
        for q in range(EMB // 16):
            c_v[r, pl.ds(16 * q, 16)] = zero
    for d in idx_cp:
        d.wait()

    b_cp = []
    catj = []
    for k in range(KCH):
        rows = pl.ds(k * CH, CH)
        b_cp.append(pltpu.async_copy(
            block_hbm.at[bid_v.at[k]], b_v.at[rows], sem_b))
        for j in range(NCATE):
            catj.append(pltpu.async_copy(
                cat_hbm.at[cid_v.at[j, k]], c_v.at[rows], sem_c, add=True))
    for d in b_cp:
        d.wait()
    pltpu.sync_copy(b_v, out_b.at[pl.ds(base, BPW)])
    for d in catj:
        d.wait()
    pltpu.sync_copy(c_v, out_c.at[pl.ds(base, BPW)])


def _u_body(users2, uid2, out_u, uid_v, g_v, ublk, sem_i, sem_u, sem_o):
    c = lax.axis_index("c")
    s = lax.axis_index("s")
    w = s * NC + c
    base = w * BPW

    pltpu.async_copy(uid2.at[pl.ds(w * RPW, RPW)], uid_v, sem_i).wait()
    for gg in range(BPW // 16):
        f = 16 * gg
        g_v[f // UCH, pl.ds(f % UCH, 16)] = uid_v[f // CH, pl.ds(f % CH, 16)]

    u_cp = []
    for k in range(2):
        u_cp.append(pltpu.async_copy(
            users2.at[g_v.at[k]], ublk.at[k % 2], sem_u))
    o_cp = []
    for k in range(UKCH):
        u_cp[k].wait()
        o_cp.append(pltpu.async_copy(
            ublk.at[k % 2], out_u.at[pl.ds(base + k * UCH, UCH)], sem_o))
        if k + 2 < UKCH:
            o_cp[k].wait()
            u_cp.append(pltpu.async_copy(
                users2.at[g_v.at[k + 2]], ublk.at[k % 2], sem_u))
    for k in range(UKCH - 2, UKCH):
        o_cp[k].wait()


_PARAMS = pltpu.CompilerParams(use_tc_tiling_on_sc=False,
                               needs_layout_passes=False)


@jax.jit
def _din_sc(users2, uid2, bid2, cid2, block_table, cat_scaled):
    mesh = plsc.VectorSubcoreMesh(core_axis_name="c", subcore_axis_name="s",
                                  num_cores=NC, num_subcores=NS)
    out_t = jax.ShapeDtypeStruct((B, EMB), jnp.float32)
    b, cc = pl.kernel(
        _bc_body,
        out_type=(out_t, out_t),
        mesh=mesh,
        compiler_params=_PARAMS,
        scratch_types=[
            pltpu.VMEM((KCH, CH), jnp.int32),
            pltpu.VMEM((NCATE, KCH, CH), jnp.int32),
            pltpu.VMEM((BPW, EMB), jnp.float32),
            pltpu.VMEM((BPW, EMB), jnp.float32),
            pltpu.SemaphoreType.DMA,
            pltpu.SemaphoreType.DMA,
            pltpu.SemaphoreType.DMA,
        ],
    )(bid2, cid2, block_table, cat_scaled)
    u128 = pl.kernel(
        _u_body,
        out_type=jax.ShapeDtypeStruct((B, 2 * EMB), jnp.float32),
        mesh=mesh,
        compiler_params=_PARAMS,
        scratch_types=[
            pltpu.VMEM((KCH, CH), jnp.int32),
            pltpu.VMEM((UKCH, UCH), jnp.int32),
            pltpu.VMEM((2, UCH, 2 * EMB), jnp.float32),
            pltpu.SemaphoreType.DMA,
            pltpu.SemaphoreType.DMA,
            pltpu.SemaphoreType.DMA,
        ],
    )(users2, uid2)
    return u128, b, cc


def kernel(user_id, block_id, cate_idx, users_table, block_table,
           category_table):
    users2 = jnp.pad(users_table, ((0, 0), (0, EMB)))
    uid2 = user_id.astype(jnp.int32).reshape(B // CH, CH)
    bid2 = block_id.astype(jnp.int32).reshape(B // CH, CH)
    cid2 = cate_idx.astype(jnp.int32).T.reshape(NCATE * (B // CH), CH)
    cat_scaled = category_table * (1.0 / NCATE)
    u128, b, cc = _din_sc(users2, uid2, bid2, cid2, block_table, cat_scaled)
    return jnp.concatenate([u128[:, :EMB], b, cc], axis=1)
